# Initial kernel scaffold; baseline (speedup 1.0000x reference)
import functools

import jax
import jax.numpy as jnp
from jax import lax
from jax.experimental import pallas as pl
from jax.experimental.pallas import tpu as pltpu
from jax.experimental.pallas import tpu_sc as plsc

NC = 2
NS = 16
NW = NC * NS
K = 100


def _sc_mesh():
    return plsc.VectorSubcoreMesh(core_axis_name="c", subcore_axis_name="s")


def _make_deg_kernel(N, E):
    ch = E // (NW * K)

    @functools.partial(
        pl.kernel,
        out_type=jax.ShapeDtypeStruct((NC, N, 16), jnp.float32),
        mesh=_sc_mesh(),
        scratch_types=[
            pltpu.VMEM((ch, K), jnp.int32),
            pltpu.VMEM((K, 16), jnp.float32),
            pltpu.VMEM_SHARED((N, 16), jnp.float32),
        ],
    )
    def deg_kernel(dst_hbm, ones_hbm, zeros_hbm, out_hbm, didx, ones_v, dacc):
        cid = lax.axis_index("c")
        sid = lax.axis_index("s")
        wid = cid * NS + sid
        rpt = N // NS
        pltpu.sync_copy(dst_hbm.at[pl.ds(wid * ch, ch)], didx)
        pltpu.sync_copy(ones_hbm, ones_v)
        pltpu.sync_copy(zeros_hbm.at[pl.ds(sid * rpt, rpt)],
                        dacc.at[pl.ds(sid * rpt, rpt)])
        plsc.subcore_barrier()

        def step(c, carry):
            pltpu.sync_copy(ones_v, dacc.at[didx.at[c]], add=True)
            return carry

        lax.fori_loop(0, ch, step, 0)
        plsc.subcore_barrier()
        pltpu.sync_copy(dacc.at[pl.ds(sid * rpt, rpt)],
                        out_hbm.at[cid, pl.ds(sid * rpt, rpt)])

    return deg_kernel


def _make_agg_kernel(N, E, H):
    ch = E // (NW * K)

    @functools.partial(
        pl.kernel,
        out_type=jax.ShapeDtypeStruct((NC, N, H), jnp.float32),
        mesh=_sc_mesh(),
        scratch_types=[
            pltpu.VMEM((ch, K), jnp.int32),
            pltpu.VMEM((ch, K), jnp.int32),
            pltpu.VMEM((K, H), jnp.float32),
            pltpu.VMEM_SHARED((N, H), jnp.float32),
        ],
    )
    def agg_kernel(y_hbm, src_hbm, dst_hbm, zeros_hbm, out_hbm,
                   sidx, didx, rows, acc):
        cid = lax.axis_index("c")
        sid = lax.axis_index("s")
        wid = cid * NS + sid
        rpt = N // NS
        pltpu.sync_copy(src_hbm.at[pl.ds(wid * ch, ch)], sidx)
        pltpu.sync_copy(dst_hbm.at[pl.ds(wid * ch, ch)], didx)
        pltpu.sync_copy(zeros_hbm.at[pl.ds(sid * rpt, rpt)],
                        acc.at[pl.ds(sid * rpt, rpt)])
        plsc.subcore_barrier()

        def step(c, carry):
            pltpu.sync_copy(y_hbm.at[sidx.at[c]], rows)
            pltpu.sync_copy(rows, acc.at[didx.at[c]], add=True)
            return carry

        lax.fori_loop(0, ch, step, 0)
        plsc.subcore_barrier()
        pltpu.sync_copy(acc.at[pl.ds(sid * rpt, rpt)],
                        out_hbm.at[cid, pl.ds(sid * rpt, rpt)])

    return agg_kernel


def _row_specs(R, *shapes):
    specs = []
    for s in shapes:
        if s[0] == "rows2":
            specs.append(pl.BlockSpec((R, s[1]), lambda i: (i, 0)))
        elif s[0] == "core3":
            c = s[2]
            specs.append(
                pl.BlockSpec((1, R, s[1]), lambda i, c=c: (c, i, 0)))
        else:
            specs.append(pl.BlockSpec(s[1], lambda i: tuple(0 for _ in s[1])))
    return specs


def _dinv_block(d0, d1):
    deg = 1.0 + d0[0, :, 0:1] + d1[0, :, 0:1]
    return lax.rsqrt(deg)


def _tc_pre(x, W, degs, R):
    N, D = x.shape
    H = W.shape[1]

    def body(d0, d1, x_ref, w_ref, y_ref):
        dinv = _dinv_block(d0[...], d1[...])
        xw = jnp.dot(x_ref[...], w_ref[...],
                     preferred_element_type=jnp.float32)
        y_ref[...] = dinv * xw

    return pl.pallas_call(
        body,
        grid=(N // R,),
        in_specs=_row_specs(R, ("core3", 16, 0), ("core3", 16, 1),
                            ("rows2", D), ("full", (D, H))),
        out_specs=pl.BlockSpec((R, H), lambda i: (i, 0)),
        out_shape=jax.ShapeDtypeStruct((N, H), jnp.float32),
    )(degs, degs, x, W)


def _tc_mid(P, y, degs, b, g, be, Wn, R):
    N, H = y.shape
    Hn = Wn.shape[1]

    def body(d0, d1, p0, p1, y_ref, b_ref, g_ref, be_ref, w_ref, o_ref):
        dinv = _dinv_block(d0[...], d1[...])
        h = dinv * (p0[0] + p1[0] + y_ref[...]) + b_ref[...]
        h = jnp.maximum(h, 0.0)
        mu = jnp.mean(h, axis=-1, keepdims=True)
        var = jnp.mean((h - mu) ** 2, axis=-1, keepdims=True)
        h = (h - mu) * lax.rsqrt(var + 1e-5) * g_ref[...] + be_ref[...]
        o_ref[...] = dinv * jnp.dot(h, w_ref[...],
                                    preferred_element_type=jnp.float32)

    return pl.pallas_call(
        body,
        grid=(N // R,),
        in_specs=_row_specs(R, ("core3", 16, 0), ("core3", 16, 1),
                            ("core3", H, 0), ("core3", H, 1),
                            ("rows2", H), ("full", (1, H)), ("full", (1, H)),
                            ("full", (1, H)), ("full", (H, Hn))),
        out_specs=pl.BlockSpec((R, Hn), lambda i: (i, 0)),
        out_shape=jax.ShapeDtypeStruct((N, Hn), jnp.float32),
    )(degs, degs, P, P, y, b.reshape(1, -1), g.reshape(1, -1),
      be.reshape(1, -1), Wn)


def _tc_final(P, y, degs, b3, Wa, ba, Wb, bb, R):
    N, H = y.shape
    O = Wb.shape[1]

    def body(d0, d1, p0, p1, y_ref, b_ref, wa_ref, ba_ref, wb_ref, bb_ref,
             o_ref):
        dinv = _dinv_block(d0[...], d1[...])
        h = dinv * (p0[0] + p1[0] + y_ref[...]) + b_ref[...]
        h = jnp.maximum(h, 0.0)
        t = jnp.dot(h, wa_ref[...],
                    preferred_element_type=jnp.float32) + ba_ref[...]
        u = jnp.dot(t, wb_ref[...],
                    preferred_element_type=jnp.float32) + bb_ref[...]
        o_ref[...] = jax.nn.sigmoid(u)

    return pl.pallas_call(
        body,
        grid=(N // R,),
        in_specs=_row_specs(R, ("core3", 16, 0), ("core3", 16, 1),
                            ("core3", H, 0), ("core3", H, 1),
                            ("rows2", H), ("full", (1, H)),
                            ("full", (H, H)), ("full", (1, H)),
                            ("full", (H, O)), ("full", (1, O))),
        out_specs=pl.BlockSpec((R, O), lambda i: (i, 0)),
        out_shape=jax.ShapeDtypeStruct((N, O), jnp.float32),
    )(degs, degs, P, P, y, b3.reshape(1, -1), Wa, ba.reshape(1, -1),
      Wb, bb.reshape(1, -1))


def kernel(x, edge_index, batch, W1, b1, W2, b2, W3, b3,
           g1, be1, g2, be2, Wa, ba, Wb, bb):
    N, D = x.shape
    E = edge_index.shape[1]
    H = W1.shape[1]
    assert E % (NW * K) == 0 and N % NS == 0

    src2d = edge_index[0].astype(jnp.int32).reshape(E // K, K)
    dst2d = edge_index[1].astype(jnp.int32).reshape(E // K, K)
    zeros_nh = jnp.zeros((N, H), jnp.float32)
    zeros_n16 = jnp.zeros((N, 16), jnp.float32)
    ones_k16 = jnp.ones((K, 16), jnp.float32)

    R = 2000
    deg_k = _make_deg_kernel(N, E)
    agg_k = _make_agg_kernel(N, E, H)

    degs = deg_k(dst2d, ones_k16, zeros_n16)
    y1 = _tc_pre(x, W1, degs, R)
    P1 = agg_k(y1, src2d, dst2d, zeros_nh)
    y2 = _tc_mid(P1, y1, degs, b1, g1, be1, W2, R)
    P2 = agg_k(y2, src2d, dst2d, zeros_nh)
    y3 = _tc_mid(P2, y2, degs, b2, g2, be2, W3, R)
    P3 = agg_k(y3, src2d, dst2d, zeros_nh)
    return _tc_final(P3, y3, degs, b3, Wa, ba, Wb, bb, R)


# speedup vs baseline: 16.7285x; 16.7285x over previous
import functools

import jax
import jax.numpy as jnp
from jax import lax
from jax.experimental import pallas as pl
from jax.experimental.pallas import tpu as pltpu
from jax.experimental.pallas import tpu_sc as plsc

NC = 2
NS = 16
NW = NC * NS
K = 100


def _sc_mesh():
    return plsc.VectorSubcoreMesh(core_axis_name="c", subcore_axis_name="s")


def _make_deg_kernel(N, E, H):
    ch = E // (NW * K)

    @functools.partial(
        pl.kernel,
        out_type=jax.ShapeDtypeStruct((NC, N, H), jnp.float32),
        mesh=_sc_mesh(),
        scratch_types=[
            pltpu.VMEM((ch, K), jnp.int32),
            pltpu.VMEM((K, H), jnp.float32),
            pltpu.VMEM_SHARED((N, H), jnp.float32),
        ],
    )
    def deg_kernel(dst_hbm, ones_hbm, zeros_hbm, out_hbm, didx, ones_v, dacc):
        cid = lax.axis_index("c")
        sid = lax.axis_index("s")
        wid = cid * NS + sid
        rpt = (N // NS) // 8 * 8
        tail = N - rpt * NS
        pltpu.sync_copy(dst_hbm.at[wid], didx)
        pltpu.sync_copy(ones_hbm, ones_v)
        pltpu.sync_copy(zeros_hbm.at[pl.ds(sid * rpt, rpt)],
                        dacc.at[pl.ds(sid * rpt, rpt)])
        if tail:
            @pl.when(sid == 0)
            def _():
                pltpu.sync_copy(zeros_hbm.at[pl.ds(rpt * NS, tail)],
                                dacc.at[pl.ds(rpt * NS, tail)])
        plsc.subcore_barrier()

        def step(c, carry):
            pltpu.sync_copy(ones_v, dacc.at[didx.at[c]], add=True)
            return carry

        lax.fori_loop(0, ch, step, 0)
        plsc.subcore_barrier()
        pltpu.sync_copy(dacc.at[pl.ds(sid * rpt, rpt)],
                        out_hbm.at[cid, pl.ds(sid * rpt, rpt)])
        if tail:
            @pl.when(sid == 0)
            def _():
                pltpu.sync_copy(dacc.at[pl.ds(rpt * NS, tail)],
                                out_hbm.at[cid, pl.ds(rpt * NS, tail)])

    return deg_kernel


def _make_agg_kernel(N, E, H):
    ch = E // (NW * K)

    @functools.partial(
        pl.kernel,
        out_type=jax.ShapeDtypeStruct((NC, N, H), jnp.float32),
        mesh=_sc_mesh(),
        scratch_types=[
            pltpu.VMEM((ch, K), jnp.int32),
            pltpu.VMEM((ch, K), jnp.int32),
            pltpu.VMEM((K, H), jnp.float32),
            pltpu.VMEM_SHARED((N, H), jnp.float32),
        ],
    )
    def agg_kernel(y_hbm, src_hbm, dst_hbm, zeros_hbm, out_hbm,
                   sidx, didx, rows, acc):
        cid = lax.axis_index("c")
        sid = lax.axis_index("s")
        wid = cid * NS + sid
        rpt = (N // NS) // 8 * 8
        tail = N - rpt * NS
        pltpu.sync_copy(src_hbm.at[wid], sidx)
        pltpu.sync_copy(dst_hbm.at[wid], didx)
        pltpu.sync_copy(zeros_hbm.at[pl.ds(sid * rpt, rpt)],
                        acc.at[pl.ds(sid * rpt, rpt)])
        if tail:
            @pl.when(sid == 0)
            def _():
                pltpu.sync_copy(zeros_hbm.at[pl.ds(rpt * NS, tail)],
                                acc.at[pl.ds(rpt * NS, tail)])
        plsc.subcore_barrier()

        def step(c, carry):
            pltpu.sync_copy(y_hbm.at[sidx.at[c]], rows)
            pltpu.sync_copy(rows, acc.at[didx.at[c]], add=True)
            return carry

        lax.fori_loop(0, ch, step, 0)
        plsc.subcore_barrier()
        pltpu.sync_copy(acc.at[pl.ds(sid * rpt, rpt)],
                        out_hbm.at[cid, pl.ds(sid * rpt, rpt)])
        if tail:
            @pl.when(sid == 0)
            def _():
                pltpu.sync_copy(acc.at[pl.ds(rpt * NS, tail)],
                                out_hbm.at[cid, pl.ds(rpt * NS, tail)])

    return agg_kernel


def _row_specs(R, *shapes):
    specs = []
    for s in shapes:
        if s[0] == "rows2":
            specs.append(pl.BlockSpec((R, s[1]), lambda i: (i, 0)))
        elif s[0] == "core3":
            c = s[2]
            specs.append(
                pl.BlockSpec((1, R, s[1]), lambda i, c=c: (c, i, 0)))
        else:
            specs.append(pl.BlockSpec(s[1], lambda i: tuple(0 for _ in s[1])))
    return specs


def _dinv_block(d0, d1):
    deg = 1.0 + d0[0, :, 0:1] + d1[0, :, 0:1]
    return lax.rsqrt(deg)


def _tc_pre(x, W, degs, R):
    N, D = x.shape
    H = W.shape[1]

    def body(d0, d1, x_ref, w_ref, y_ref):
        dinv = _dinv_block(d0[...], d1[...])
        xw = jnp.dot(x_ref[...], w_ref[...],
                     preferred_element_type=jnp.float32)
        y_ref[...] = dinv * xw

    return pl.pallas_call(
        body,
        grid=(N // R,),
        in_specs=_row_specs(R, ("core3", H, 0), ("core3", H, 1),
                            ("rows2", D), ("full", (D, H))),
        out_specs=pl.BlockSpec((R, H), lambda i: (i, 0)),
        out_shape=jax.ShapeDtypeStruct((N, H), jnp.float32),
    )(degs, degs, x, W)


def _tc_mid(P, y, degs, b, g, be, Wn, R):
    N, H = y.shape
    Hn = Wn.shape[1]

    def body(d0, d1, p0, p1, y_ref, b_ref, g_ref, be_ref, w_ref, o_ref):
        dinv = _dinv_block(d0[...], d1[...])
        h = dinv * (p0[0] + p1[0] + y_ref[...]) + b_ref[...]
        h = jnp.maximum(h, 0.0)
        mu = jnp.mean(h, axis=-1, keepdims=True)
        var = jnp.mean((h - mu) ** 2, axis=-1, keepdims=True)
        h = (h - mu) * lax.rsqrt(var + 1e-5) * g_ref[...] + be_ref[...]
        o_ref[...] = dinv * jnp.dot(h, w_ref[...],
                                    preferred_element_type=jnp.float32)

    return pl.pallas_call(
        body,
        grid=(N // R,),
        in_specs=_row_specs(R, ("core3", H, 0), ("core3", H, 1),
                            ("core3", H, 0), ("core3", H, 1),
                            ("rows2", H), ("full", (1, H)), ("full", (1, H)),
                            ("full", (1, H)), ("full", (H, Hn))),
        out_specs=pl.BlockSpec((R, Hn), lambda i: (i, 0)),
        out_shape=jax.ShapeDtypeStruct((N, Hn), jnp.float32),
    )(degs, degs, P, P, y, b.reshape(1, -1), g.reshape(1, -1),
      be.reshape(1, -1), Wn)


def _tc_final(P, y, degs, b3, Wa, ba, Wb, bb, R):
    N, H = y.shape
    O = Wb.shape[1]

    def body(d0, d1, p0, p1, y_ref, b_ref, wa_ref, ba_ref, wb_ref, bb_ref,
             o_ref):
        dinv = _dinv_block(d0[...], d1[...])
        h = dinv * (p0[0] + p1[0] + y_ref[...]) + b_ref[...]
        h = jnp.maximum(h, 0.0)
        t = jnp.dot(h, wa_ref[...],
                    preferred_element_type=jnp.float32) + ba_ref[...]
        u = jnp.dot(t, wb_ref[...],
                    preferred_element_type=jnp.float32) + bb_ref[...]
        o_ref[...] = jax.nn.sigmoid(u)

    return pl.pallas_call(
        body,
        grid=(N // R,),
        in_specs=_row_specs(R, ("core3", H, 0), ("core3", H, 1),
                            ("core3", H, 0), ("core3", H, 1),
                            ("rows2", H), ("full", (1, H)),
                            ("full", (H, H)), ("full", (1, H)),
                            ("full", (H, O)), ("full", (1, O))),
        out_specs=pl.BlockSpec((R, O), lambda i: (i, 0)),
        out_shape=jax.ShapeDtypeStruct((N, O), jnp.float32),
    )(degs, degs, P, P, y, b3.reshape(1, -1), Wa, ba.reshape(1, -1),
      Wb, bb.reshape(1, -1))


def kernel(x, edge_index, batch, W1, b1, W2, b2, W3, b3,
           g1, be1, g2, be2, Wa, ba, Wb, bb):
    N, D = x.shape
    E = edge_index.shape[1]
    H = W1.shape[1]
    assert E % (NW * K) == 0 and N % NS == 0

    ch = E // (NW * K)
    src3d = edge_index[0].astype(jnp.int32).reshape(NW, ch, K)
    dst3d = edge_index[1].astype(jnp.int32).reshape(NW, ch, K)
    zeros_nh = jnp.zeros((N, H), jnp.float32)
    ones_kh = jnp.ones((K, H), jnp.float32)

    R = 2000
    deg_k = _make_deg_kernel(N, E, H)
    agg_k = _make_agg_kernel(N, E, H)

    degs = deg_k(dst3d, ones_kh, zeros_nh)
    y1 = _tc_pre(x, W1, degs, R)
    P1 = agg_k(y1, src3d, dst3d, zeros_nh)
    y2 = _tc_mid(P1, y1, degs, b1, g1, be1, W2, R)
    P2 = agg_k(y2, src3d, dst3d, zeros_nh)
    y3 = _tc_mid(P2, y2, degs, b2, g2, be2, W3, R)
    P3 = agg_k(y3, src3d, dst3d, zeros_nh)
    return _tc_final(P3, y3, degs, b3, Wa, ba, Wb, bb, R)


# speedup vs baseline: 22.2986x; 1.3330x over previous
import functools

import jax
import jax.numpy as jnp
from jax import lax
from jax.experimental import pallas as pl
from jax.experimental.pallas import tpu as pltpu
from jax.experimental.pallas import tpu_sc as plsc

NC = 2
NS = 16
NW = NC * NS
K = 125


def _sc_mesh():
    return plsc.VectorSubcoreMesh(core_axis_name="c", subcore_axis_name="s")


def _make_deg_kernel(N, E, H):
    ch = E // (NW * K)

    @functools.partial(
        pl.kernel,
        out_type=jax.ShapeDtypeStruct((NC, N, H), jnp.float32),
        mesh=_sc_mesh(),
        scratch_types=[
            pltpu.VMEM((ch, K), jnp.int32),
            pltpu.VMEM((K, H), jnp.float32),
            pltpu.VMEM_SHARED((N, H), jnp.float32),
            pltpu.SemaphoreType.DMA,
        ],
    )
    def deg_kernel(dst_hbm, ones_hbm, zeros_hbm, out_hbm, didx, ones_v, dacc,
                   sem):
        cid = lax.axis_index("c")
        sid = lax.axis_index("s")
        wid = cid * NS + sid
        rpt = (N // NS) // 8 * 8
        tail = N - rpt * NS
        pltpu.sync_copy(dst_hbm.at[wid], didx)
        pltpu.sync_copy(ones_hbm, ones_v)
        pltpu.sync_copy(zeros_hbm.at[pl.ds(sid * rpt, rpt)],
                        dacc.at[pl.ds(sid * rpt, rpt)])
        if tail:
            @pl.when(sid == 0)
            def _():
                pltpu.sync_copy(zeros_hbm.at[pl.ds(rpt * NS, tail)],
                                dacc.at[pl.ds(rpt * NS, tail)])
        plsc.subcore_barrier()

        def fire(c, carry):
            pltpu.async_copy(ones_v, dacc.at[didx.at[c]], sem, add=True)
            return carry

        lax.fori_loop(0, ch, fire, 0)

        def drain(c, carry):
            pltpu.make_async_copy(ones_v, dacc.at[didx.at[0]], sem).wait()
            return carry

        lax.fori_loop(0, ch, drain, 0)
        plsc.subcore_barrier()
        pltpu.sync_copy(dacc.at[pl.ds(sid * rpt, rpt)],
                        out_hbm.at[cid, pl.ds(sid * rpt, rpt)])
        if tail:
            @pl.when(sid == 0)
            def _():
                pltpu.sync_copy(dacc.at[pl.ds(rpt * NS, tail)],
                                out_hbm.at[cid, pl.ds(rpt * NS, tail)])

    return deg_kernel


def _make_agg_kernel(N, E, H):
    ch = E // (NW * K)

    @functools.partial(
        pl.kernel,
        out_type=jax.ShapeDtypeStruct((NC, N, H), jnp.float32),
        mesh=_sc_mesh(),
        scratch_types=[
            pltpu.VMEM((ch, K), jnp.int32),
            pltpu.VMEM((8, K), jnp.int32),
            pltpu.VMEM((8, K), jnp.int32),
            pltpu.VMEM((K, H), jnp.float32),
            pltpu.VMEM((K, H), jnp.float32),
            pltpu.VMEM_SHARED((N, H), jnp.float32),
            pltpu.SemaphoreType.DMA,
            pltpu.SemaphoreType.DMA,
            pltpu.SemaphoreType.DMA,
            pltpu.SemaphoreType.DMA,
            pltpu.SemaphoreType.DMA,
            pltpu.SemaphoreType.DMA,
        ],
    )
    def agg_kernel(y_hbm, src_hbm, dst4d_hbm, zeros_hbm, out_hbm,
                   sidx, didx0, didx1, rows0, rows1, acc,
                   i0, i1, g0, g1, s0, s1):
        cid = lax.axis_index("c")
        sid = lax.axis_index("s")
        wid = cid * NS + sid
        rpt = (N // NS) // 8 * 8
        tail = N - rpt * NS
        pltpu.sync_copy(src_hbm.at[wid], sidx)
        pltpu.sync_copy(zeros_hbm.at[pl.ds(sid * rpt, rpt)],
                        acc.at[pl.ds(sid * rpt, rpt)])
        if tail:
            @pl.when(sid == 0)
            def _():
                pltpu.sync_copy(zeros_hbm.at[pl.ds(rpt * NS, tail)],
                                acc.at[pl.ds(rpt * NS, tail)])
        plsc.subcore_barrier()

        didx = (didx0, didx1)
        rows = (rows0, rows1)
        isem = (i0, i1)
        gsem = (g0, g1)
        ssem = (s0, s1)

        def pair(p, carry):
            for b in (0, 1):
                c = 2 * p + b

                @pl.when(c >= 2)
                def _():
                    pltpu.make_async_copy(
                        rows[b], acc.at[didx[b].at[0]], ssem[b]).wait()

                pltpu.async_copy(dst4d_hbm.at[wid, c],
                                 didx[b].at[pl.ds(0, 1)], isem[b])
                pltpu.async_copy(y_hbm.at[sidx.at[c]], rows[b], gsem[b])
                pltpu.make_async_copy(dst4d_hbm.at[wid, c],
                                      didx[b].at[pl.ds(0, 1)], isem[b]).wait()
                pltpu.make_async_copy(
                    y_hbm.at[sidx.at[c]], rows[b], gsem[b]).wait()
                pltpu.async_copy(
                    rows[b], acc.at[didx[b].at[0]], ssem[b], add=True)
            return carry

        lax.fori_loop(0, ch // 2, pair, 0)
        for b in (0, 1):
            pltpu.make_async_copy(
                rows[b], acc.at[didx[b].at[0]], ssem[b]).wait()
        plsc.subcore_barrier()
        pltpu.sync_copy(acc.at[pl.ds(sid * rpt, rpt)],
                        out_hbm.at[cid, pl.ds(sid * rpt, rpt)])
        if tail:
            @pl.when(sid == 0)
            def _():
                pltpu.sync_copy(acc.at[pl.ds(rpt * NS, tail)],
                                out_hbm.at[cid, pl.ds(rpt * NS, tail)])

    return agg_kernel


def _row_specs(R, *shapes):
    specs = []
    for s in shapes:
        if s[0] == "rows2":
            specs.append(pl.BlockSpec((R, s[1]), lambda i: (i, 0)))
        elif s[0] == "core3":
            c = s[2]
            specs.append(
                pl.BlockSpec((1, R, s[1]), lambda i, c=c: (c, i, 0)))
        else:
            specs.append(pl.BlockSpec(s[1], lambda i: tuple(0 for _ in s[1])))
    return specs


def _dinv_block(d0, d1):
    deg = 1.0 + d0[0, :, 0:1] + d1[0, :, 0:1]
    return lax.rsqrt(deg)


def _tc_pre(x, W, degs, R):
    N, D = x.shape
    H = W.shape[1]

    def body(d0, d1, x_ref, w_ref, y_ref):
        dinv = _dinv_block(d0[...], d1[...])
        xw = jnp.dot(x_ref[...], w_ref[...],
                     preferred_element_type=jnp.float32)
        y_ref[...] = dinv * xw

    return pl.pallas_call(
        body,
        grid=(N // R,),
        in_specs=_row_specs(R, ("core3", H, 0), ("core3", H, 1),
                            ("rows2", D), ("full", (D, H))),
        out_specs=pl.BlockSpec((R, H), lambda i: (i, 0)),
        out_shape=jax.ShapeDtypeStruct((N, H), jnp.float32),
    )(degs, degs, x, W)


def _tc_mid(P, y, degs, b, g, be, Wn, R):
    N, H = y.shape
    Hn = Wn.shape[1]

    def body(d0, d1, p0, p1, y_ref, b_ref, g_ref, be_ref, w_ref, o_ref):
        dinv = _dinv_block(d0[...], d1[...])
        h = dinv * (p0[0] + p1[0] + y_ref[...]) + b_ref[...]
        h = jnp.maximum(h, 0.0)
        mu = jnp.mean(h, axis=-1, keepdims=True)
        var = jnp.mean((h - mu) ** 2, axis=-1, keepdims=True)
        h = (h - mu) * lax.rsqrt(var + 1e-5) * g_ref[...] + be_ref[...]
        o_ref[...] = dinv * jnp.dot(h, w_ref[...],
                                    preferred_element_type=jnp.float32)

    return pl.pallas_call(
        body,
        grid=(N // R,),
        in_specs=_row_specs(R, ("core3", H, 0), ("core3", H, 1),
                            ("core3", H, 0), ("core3", H, 1),
                            ("rows2", H), ("full", (1, H)), ("full", (1, H)),
                            ("full", (1, H)), ("full", (H, Hn))),
        out_specs=pl.BlockSpec((R, Hn), lambda i: (i, 0)),
        out_shape=jax.ShapeDtypeStruct((N, Hn), jnp.float32),
    )(degs, degs, P, P, y, b.reshape(1, -1), g.reshape(1, -1),
      be.reshape(1, -1), Wn)


def _tc_final(P, y, degs, b3, Wa, ba, Wb, bb, R):
    N, H = y.shape
    O = Wb.shape[1]

    def body(d0, d1, p0, p1, y_ref, b_ref, wa_ref, ba_ref, wb_ref, bb_ref,
             o_ref):
        dinv = _dinv_block(d0[...], d1[...])
        h = dinv * (p0[0] + p1[0] + y_ref[...]) + b_ref[...]
        h = jnp.maximum(h, 0.0)
        t = jnp.dot(h, wa_ref[...],
                    preferred_element_type=jnp.float32) + ba_ref[...]
        u = jnp.dot(t, wb_ref[...],
                    preferred_element_type=jnp.float32) + bb_ref[...]
        o_ref[...] = jax.nn.sigmoid(u)

    return pl.pallas_call(
        body,
        grid=(N // R,),
        in_specs=_row_specs(R, ("core3", H, 0), ("core3", H, 1),
                            ("core3", H, 0), ("core3", H, 1),
                            ("rows2", H), ("full", (1, H)),
                            ("full", (H, H)), ("full", (1, H)),
                            ("full", (H, O)), ("full", (1, O))),
        out_specs=pl.BlockSpec((R, O), lambda i: (i, 0)),
        out_shape=jax.ShapeDtypeStruct((N, O), jnp.float32),
    )(degs, degs, P, P, y, b3.reshape(1, -1), Wa, ba.reshape(1, -1),
      Wb, bb.reshape(1, -1))


def kernel(x, edge_index, batch, W1, b1, W2, b2, W3, b3,
           g1, be1, g2, be2, Wa, ba, Wb, bb):
    N, D = x.shape
    E = edge_index.shape[1]
    H = W1.shape[1]
    assert E % (NW * K) == 0 and N % NS == 0

    ch = E // (NW * K)
    src3d = edge_index[0].astype(jnp.int32).reshape(NW, ch, K)
    dst3d = edge_index[1].astype(jnp.int32).reshape(NW, ch, K)
    dst4d = dst3d.reshape(NW, ch, 1, K)
    zeros_nh = jnp.zeros((N, H), jnp.float32)
    ones_kh = jnp.ones((K, H), jnp.float32)

    R = 2000
    deg_k = _make_deg_kernel(N, E, H)
    agg_k = _make_agg_kernel(N, E, H)

    degs = deg_k(dst3d, ones_kh, zeros_nh)
    y1 = _tc_pre(x, W1, degs, R)
    P1 = agg_k(y1, src3d, dst4d, zeros_nh)
    y2 = _tc_mid(P1, y1, degs, b1, g1, be1, W2, R)
    P2 = agg_k(y2, src3d, dst4d, zeros_nh)
    y3 = _tc_mid(P2, y2, degs, b2, g2, be2, W3, R)
    P3 = agg_k(y3, src3d, dst4d, zeros_nh)
    return _tc_final(P3, y3, degs, b3, Wa, ba, Wb, bb, R)


# speedup vs baseline: 22.4384x; 1.0063x over previous
import functools

import jax
import jax.numpy as jnp
from jax import lax
from jax.experimental import pallas as pl
from jax.experimental.pallas import tpu as pltpu
from jax.experimental.pallas import tpu_sc as plsc

NC = 2
NS = 16
NW = NC * NS
K = 125


def _sc_mesh():
    return plsc.VectorSubcoreMesh(core_axis_name="c", subcore_axis_name="s")


def _make_deg_kernel(N, E, H):
    ch = E // (NW * K)

    @functools.partial(
        pl.kernel,
        out_type=jax.ShapeDtypeStruct((NC, N, H), jnp.float32),
        mesh=_sc_mesh(),
        scratch_types=[
            pltpu.VMEM((ch, K), jnp.int32),
            pltpu.VMEM((K, H), jnp.float32),
            pltpu.VMEM_SHARED((N, H), jnp.float32),
            pltpu.SemaphoreType.DMA,
        ],
    )
    def deg_kernel(dst_hbm, ones_hbm, zeros_hbm, out_hbm, didx, ones_v, dacc,
                   sem):
        cid = lax.axis_index("c")
        sid = lax.axis_index("s")
        wid = cid * NS + sid
        rpt = (N // NS) // 8 * 8
        tail = N - rpt * NS
        pltpu.sync_copy(dst_hbm.at[wid], didx)
        pltpu.sync_copy(ones_hbm, ones_v)
        pltpu.sync_copy(zeros_hbm.at[pl.ds(sid * rpt, rpt)],
                        dacc.at[pl.ds(sid * rpt, rpt)])
        if tail:
            @pl.when(sid == 0)
            def _():
                pltpu.sync_copy(zeros_hbm.at[pl.ds(rpt * NS, tail)],
                                dacc.at[pl.ds(rpt * NS, tail)])
        plsc.subcore_barrier()

        def fire(c, carry):
            pltpu.async_copy(ones_v, dacc.at[didx.at[c]], sem, add=True)
            return carry

        lax.fori_loop(0, ch, fire, 0)

        def drain(c, carry):
            pltpu.make_async_copy(ones_v, dacc.at[didx.at[0]], sem).wait()
            return carry

        lax.fori_loop(0, ch, drain, 0)
        plsc.subcore_barrier()
        pltpu.sync_copy(dacc.at[pl.ds(sid * rpt, rpt)],
                        out_hbm.at[cid, pl.ds(sid * rpt, rpt)])
        if tail:
            @pl.when(sid == 0)
            def _():
                pltpu.sync_copy(dacc.at[pl.ds(rpt * NS, tail)],
                                out_hbm.at[cid, pl.ds(rpt * NS, tail)])

    return deg_kernel


def _make_agg_kernel(N, E, H):
    ch = E // (NW * K)

    @functools.partial(
        pl.kernel,
        out_type=jax.ShapeDtypeStruct((NC, N, H), jnp.float32),
        mesh=_sc_mesh(),
        scratch_types=[
            pltpu.VMEM((ch, K), jnp.int32),
            pltpu.VMEM((8, K), jnp.int32),
            pltpu.VMEM((8, K), jnp.int32),
            pltpu.VMEM((K, H), jnp.float32),
            pltpu.VMEM((K, H), jnp.float32),
            pltpu.VMEM_SHARED((N, H), jnp.float32),
            pltpu.SemaphoreType.DMA,
            pltpu.SemaphoreType.DMA,
            pltpu.SemaphoreType.DMA,
            pltpu.SemaphoreType.DMA,
            pltpu.SemaphoreType.DMA,
            pltpu.SemaphoreType.DMA,
        ],
    )
    def agg_kernel(y_hbm, src_hbm, dst4d_hbm, zeros_hbm, out_hbm,
                   sidx, didx0, didx1, rows0, rows1, acc,
                   i0, i1, g0, g1, s0, s1):
        cid = lax.axis_index("c")
        sid = lax.axis_index("s")
        wid = cid * NS + sid
        rpt = (N // NS) // 8 * 8
        tail = N - rpt * NS
        pltpu.sync_copy(src_hbm.at[wid], sidx)
        @pl.when(cid == 0)
        def _():
            pltpu.sync_copy(y_hbm.at[pl.ds(sid * rpt, rpt)],
                            acc.at[pl.ds(sid * rpt, rpt)])
        @pl.when(cid == 1)
        def _():
            pltpu.sync_copy(zeros_hbm.at[pl.ds(sid * rpt, rpt)],
                            acc.at[pl.ds(sid * rpt, rpt)])
        if tail:
            @pl.when((sid == 0) & (cid == 0))
            def _():
                pltpu.sync_copy(y_hbm.at[pl.ds(rpt * NS, tail)],
                                acc.at[pl.ds(rpt * NS, tail)])
            @pl.when((sid == 0) & (cid == 1))
            def _():
                pltpu.sync_copy(zeros_hbm.at[pl.ds(rpt * NS, tail)],
                                acc.at[pl.ds(rpt * NS, tail)])
        plsc.subcore_barrier()

        didx = (didx0, didx1)
        rows = (rows0, rows1)
        isem = (i0, i1)
        gsem = (g0, g1)
        ssem = (s0, s1)

        def pair(p, carry):
            for b in (0, 1):
                c = 2 * p + b

                @pl.when(c >= 2)
                def _():
                    pltpu.make_async_copy(
                        rows[b], acc.at[didx[b].at[0]], ssem[b]).wait()

                pltpu.async_copy(dst4d_hbm.at[wid, c],
                                 didx[b].at[pl.ds(0, 1)], isem[b])
                pltpu.async_copy(y_hbm.at[sidx.at[c]], rows[b], gsem[b])
                pltpu.make_async_copy(dst4d_hbm.at[wid, c],
                                      didx[b].at[pl.ds(0, 1)], isem[b]).wait()
                pltpu.make_async_copy(
                    y_hbm.at[sidx.at[c]], rows[b], gsem[b]).wait()
                pltpu.async_copy(
                    rows[b], acc.at[didx[b].at[0]], ssem[b], add=True)
            return carry

        lax.fori_loop(0, ch // 2, pair, 0)
        for b in (0, 1):
            pltpu.make_async_copy(
                rows[b], acc.at[didx[b].at[0]], ssem[b]).wait()
        plsc.subcore_barrier()
        pltpu.sync_copy(acc.at[pl.ds(sid * rpt, rpt)],
                        out_hbm.at[cid, pl.ds(sid * rpt, rpt)])
        if tail:
            @pl.when(sid == 0)
            def _():
                pltpu.sync_copy(acc.at[pl.ds(rpt * NS, tail)],
                                out_hbm.at[cid, pl.ds(rpt * NS, tail)])

    return agg_kernel


def _row_specs(R, *shapes):
    specs = []
    for s in shapes:
        if s[0] == "rows2":
            specs.append(pl.BlockSpec((R, s[1]), lambda i: (i, 0)))
        elif s[0] == "core3":
            c = s[2]
            specs.append(
                pl.BlockSpec((1, R, s[1]), lambda i, c=c: (c, i, 0)))
        else:
            specs.append(pl.BlockSpec(s[1], lambda i: tuple(0 for _ in s[1])))
    return specs


def _dinv_block(d0, d1):
    deg = 1.0 + d0[0, :, 0:1] + d1[0, :, 0:1]
    return lax.rsqrt(deg)


def _tc_pre(x, W, degs, R):
    N, D = x.shape
    H = W.shape[1]

    def body(d0, d1, x_ref, w_ref, y_ref):
        dinv = _dinv_block(d0[...], d1[...])
        xw = jnp.dot(x_ref[...], w_ref[...],
                     preferred_element_type=jnp.float32)
        y_ref[...] = dinv * xw

    return pl.pallas_call(
        body,
        grid=(N // R,),
        in_specs=_row_specs(R, ("core3", 8, 0), ("core3", 8, 1),
                            ("rows2", D), ("full", (D, H))),
        out_specs=pl.BlockSpec((R, H), lambda i: (i, 0)),
        out_shape=jax.ShapeDtypeStruct((N, H), jnp.float32),
    )(degs, degs, x, W)


def _tc_mid(P, degs, b, g, be, Wn, R):
    _, N, H = P.shape[0], P.shape[1], P.shape[2]
    Hn = Wn.shape[1]

    def body(d0, d1, p0, p1, b_ref, g_ref, be_ref, w_ref, o_ref):
        dinv = _dinv_block(d0[...], d1[...])
        h = dinv * (p0[0] + p1[0]) + b_ref[...]
        h = jnp.maximum(h, 0.0)
        mu = jnp.mean(h, axis=-1, keepdims=True)
        var = jnp.mean((h - mu) ** 2, axis=-1, keepdims=True)
        h = (h - mu) * lax.rsqrt(var + 1e-5) * g_ref[...] + be_ref[...]
        o_ref[...] = dinv * jnp.dot(h, w_ref[...],
                                    preferred_element_type=jnp.float32)

    return pl.pallas_call(
        body,
        grid=(N // R,),
        in_specs=_row_specs(R, ("core3", 8, 0), ("core3", 8, 1),
                            ("core3", H, 0), ("core3", H, 1),
                            ("full", (1, H)), ("full", (1, H)),
                            ("full", (1, H)), ("full", (H, Hn))),
        out_specs=pl.BlockSpec((R, Hn), lambda i: (i, 0)),
        out_shape=jax.ShapeDtypeStruct((N, Hn), jnp.float32),
    )(degs, degs, P, P, b.reshape(1, -1), g.reshape(1, -1),
      be.reshape(1, -1), Wn)


def _tc_final(P, degs, b3, Wa, ba, Wb, bb, R):
    N, H = P.shape[1], P.shape[2]
    O = Wb.shape[1]

    def body(d0, d1, p0, p1, b_ref, wa_ref, ba_ref, wb_ref, bb_ref,
             o_ref):
        dinv = _dinv_block(d0[...], d1[...])
        h = dinv * (p0[0] + p1[0]) + b_ref[...]
        h = jnp.maximum(h, 0.0)
        t = jnp.dot(h, wa_ref[...],
                    preferred_element_type=jnp.float32) + ba_ref[...]
        u = jnp.dot(t, wb_ref[...],
                    preferred_element_type=jnp.float32) + bb_ref[...]
        o_ref[...] = jax.nn.sigmoid(u)

    return pl.pallas_call(
        body,
        grid=(N // R,),
        in_specs=_row_specs(R, ("core3", 8, 0), ("core3", 8, 1),
                            ("core3", H, 0), ("core3", H, 1),
                            ("full", (1, H)),
                            ("full", (H, H)), ("full", (1, H)),
                            ("full", (H, O)), ("full", (1, O))),
        out_specs=pl.BlockSpec((R, O), lambda i: (i, 0)),
        out_shape=jax.ShapeDtypeStruct((N, O), jnp.float32),
    )(degs, degs, P, P, b3.reshape(1, -1), Wa, ba.reshape(1, -1),
      Wb, bb.reshape(1, -1))


def kernel(x, edge_index, batch, W1, b1, W2, b2, W3, b3,
           g1, be1, g2, be2, Wa, ba, Wb, bb):
    N, D = x.shape
    E = edge_index.shape[1]
    H = W1.shape[1]
    assert E % (NW * K) == 0 and N % NS == 0

    ch = E // (NW * K)
    src3d = edge_index[0].astype(jnp.int32).reshape(NW, ch, K)
    dst3d = edge_index[1].astype(jnp.int32).reshape(NW, ch, K)
    dst4d = dst3d.reshape(NW, ch, 1, K)
    zeros_nh = jnp.zeros((N, H), jnp.float32)
    ones_kh = jnp.ones((K, H), jnp.float32)

    R = 2000
    deg_k = _make_deg_kernel(N, E, H)
    agg_k = _make_agg_kernel(N, E, H)

    degs = deg_k(dst3d, ones_kh, zeros_nh)[:, :, :8]
    y1 = _tc_pre(x, W1, degs, R)
    P1 = agg_k(y1, src3d, dst4d, zeros_nh)
    y2 = _tc_mid(P1, degs, b1, g1, be1, W2, R)
    P2 = agg_k(y2, src3d, dst4d, zeros_nh)
    y3 = _tc_mid(P2, degs, b2, g2, be2, W3, R)
    P3 = agg_k(y3, src3d, dst4d, zeros_nh)
    return _tc_final(P3, degs, b3, Wa, ba, Wb, bb, R)
